# baseline (speedup 1.0000x reference)
import functools

import jax
import jax.numpy as jnp
from jax import lax
from jax.experimental import pallas as pl
from jax.experimental.pallas import tpu as pltpu
from jax.experimental.pallas import tpu_sc as plsc

N = 10000
E = 320000
C = 128
D = 16
OUT = 64

NC = 2
NS = 16
NW = NC * NS
BE = 40

SEGS = (107520, 107520, 104960)
BET = 2560

_F32 = jnp.float32
_BF16 = jnp.bfloat16


def _tables_first_body(x_ref, wd_ref, ws_ref, bd_ref, td_ref, ts_ref):
    h = x_ref[...]
    td_ref[...] = jnp.dot(h, wd_ref[...], preferred_element_type=_F32) + bd_ref[...]
    ts_ref[...] = jnp.dot(h, ws_ref[...], preferred_element_type=_F32)


def _tables_res_body(x_ref, a0_ref, a1_ref, a2_ref, a3_ref, a4_ref, a5_ref,
                     wd_ref, ws_ref, bd_ref, h_ref, td_ref, ts_ref):
    h = jnp.maximum(x_ref[...] + (a0_ref[...] + a1_ref[...]) +
                    (a2_ref[...] + a3_ref[...]) + (a4_ref[...] + a5_ref[...]), 0.0)
    h_ref[...] = h
    td_ref[...] = jnp.dot(h, wd_ref[...], preferred_element_type=_F32) + bd_ref[...]
    ts_ref[...] = jnp.dot(h, ws_ref[...], preferred_element_type=_F32)


_BN = 1000


def _tables_first(x, wd, ws, bd):
    grid = N // _BN
    return pl.pallas_call(
        _tables_first_body,
        grid=(grid,),
        in_specs=[
            pl.BlockSpec((_BN, C), lambda i: (i, 0)),
            pl.BlockSpec((C, 2 * C), lambda i: (0, 0)),
            pl.BlockSpec((C, 2 * C), lambda i: (0, 0)),
            pl.BlockSpec((1, 2 * C), lambda i: (0, 0)),
        ],
        out_specs=[
            pl.BlockSpec((_BN, 2 * C), lambda i: (i, 0)),
            pl.BlockSpec((_BN, 2 * C), lambda i: (i, 0)),
        ],
        out_shape=[
            jax.ShapeDtypeStruct((N, 2 * C), _F32),
            jax.ShapeDtypeStruct((N, 2 * C), _F32),
        ],
    )(x, wd, ws, bd)


def _tables_res(x, aggs, wd, ws, bd):
    grid = N // _BN
    return pl.pallas_call(
        _tables_res_body,
        grid=(grid,),
        in_specs=[
            pl.BlockSpec((_BN, C), lambda i: (i, 0)),
            pl.BlockSpec((_BN, C), lambda i: (i, 0)),
            pl.BlockSpec((_BN, C), lambda i: (i + N // _BN, 0)),
            pl.BlockSpec((_BN, C), lambda i: (i, 0)),
            pl.BlockSpec((_BN, C), lambda i: (i + N // _BN, 0)),
            pl.BlockSpec((_BN, C), lambda i: (i, 0)),
            pl.BlockSpec((_BN, C), lambda i: (i + N // _BN, 0)),
            pl.BlockSpec((C, 2 * C), lambda i: (0, 0)),
            pl.BlockSpec((C, 2 * C), lambda i: (0, 0)),
            pl.BlockSpec((1, 2 * C), lambda i: (0, 0)),
        ],
        out_specs=[
            pl.BlockSpec((_BN, C), lambda i: (i, 0)),
            pl.BlockSpec((_BN, 2 * C), lambda i: (i, 0)),
            pl.BlockSpec((_BN, 2 * C), lambda i: (i, 0)),
        ],
        out_shape=[
            jax.ShapeDtypeStruct((N, C), _F32),
            jax.ShapeDtypeStruct((N, 2 * C), _F32),
            jax.ShapeDtypeStruct((N, 2 * C), _F32),
        ],
    )(x, aggs[0], aggs[0], aggs[1], aggs[1], aggs[2], aggs[2], wd, ws, bd)


def _make_gather_body(ne):
    epw = ne // NW
    nchunk = epw // BE
    npair = nchunk // 2
    assert nchunk % 2 == 0 and nchunk * BE == epw and epw % 8 == 0

    def body(td_hbm, ts_hbm, dsti_hbm, srci_hbm, out_hbm,
             idxad, idxas,
             bufd0, bufs0, obuf0, bufd1, bufs1, obuf1,
             semg0, semg1, semo0, semo1):
        c = lax.axis_index("c")
        s = lax.axis_index("s")
        wid = s * NC + c
        base = wid * epw

        pltpu.sync_copy(dsti_hbm.at[pl.ds(base, epw)], idxad)
        pltpu.sync_copy(srci_hbm.at[pl.ds(base, epw)], idxas)

        ring = ((bufd0, bufs0, obuf0, semg0, semo0),
                (bufd1, bufs1, obuf1, semg1, semo1))

        for b in range(2):
            bd, bs_, ob, sg, so = ring[b]
            pltpu.async_copy(td_hbm.at[idxad.at[pl.ds(b * BE, BE)]], bd, sg)
            pltpu.async_copy(ts_hbm.at[idxas.at[pl.ds(b * BE, BE)]], bs_, sg)

        @pl.loop(0, npair)
        def _pair(t):
            for b in range(2):
                bd, bs_, ob, sg, so = ring[b]
                ch = 2 * t + b
                off = base + ch * BE
                pltpu.make_async_copy(td_hbm.at[idxad.at[pl.ds(0, BE)]], bd, sg).wait()
                pltpu.make_async_copy(ts_hbm.at[idxas.at[pl.ds(0, BE)]], bs_, sg).wait()

                @pl.when(t > 0)
                def _():
                    pltpu.make_async_copy(ob, out_hbm.at[pl.ds(off, BE)], so).wait()

                @plsc.parallel_loop(0, BE, unroll=4)
                def _row(r):
                    for k in range(2 * C // 16):
                        ob[r, pl.ds(k * 16, 16)] = (bd[r, pl.ds(k * 16, 16)] +
                                                    bs_[r, pl.ds(k * 16, 16)])

                pltpu.async_copy(ob, out_hbm.at[pl.ds(off, BE)], so)

                @pl.when(ch + 2 < nchunk)
                def _():
                    off2 = (ch + 2) * BE
                    pltpu.async_copy(td_hbm.at[idxad.at[pl.ds(off2, BE)]], bd, sg)
                    pltpu.async_copy(ts_hbm.at[idxas.at[pl.ds(off2, BE)]], bs_, sg)

        for b in range(2):
            bd, bs_, ob, sg, so = ring[b]
            pltpu.make_async_copy(ob, out_hbm.at[pl.ds(base, BE)], so).wait()

    return body, epw


def _make_sc_gather(ne):
    body, epw = _make_gather_body(ne)
    mesh = plsc.VectorSubcoreMesh(core_axis_name="c", subcore_axis_name="s")
    kern = pl.kernel(
        body,
        out_type=jax.ShapeDtypeStruct((ne, 2 * C), _F32),
        mesh=mesh,
        scratch_types=[
            pltpu.VMEM((epw,), jnp.int32),
            pltpu.VMEM((epw,), jnp.int32),
            pltpu.VMEM((BE, 2 * C), _F32),
            pltpu.VMEM((BE, 2 * C), _F32),
            pltpu.VMEM((BE, 2 * C), _F32),
            pltpu.VMEM((BE, 2 * C), _F32),
            pltpu.VMEM((BE, 2 * C), _F32),
            pltpu.VMEM((BE, 2 * C), _F32),
            pltpu.SemaphoreType.DMA,
            pltpu.SemaphoreType.DMA,
            pltpu.SemaphoreType.DMA,
            pltpu.SemaphoreType.DMA,
        ],
    )
    return kern


def _edge_body(g_ref, ewt_ref, wcat_ref, m_ref):
    ep = lax.dot_general(ewt_ref[...], wcat_ref[...],
                         dimension_numbers=(((0,), (0,)), ((), ())),
                         preferred_element_type=_F32)
    g = g_ref[...].astype(_F32)
    fpre = g[:, :C] + ep[:, :C]
    spre = g[:, C:] + ep[:, C:]
    sig = 0.5 * jnp.tanh(0.5 * fpre) + 0.5
    sp = jnp.maximum(spre, 0.0) + jnp.log1p(jnp.exp(-jnp.abs(spre)))
    m_ref[...] = sig * sp


def _edge_tc(g, ewt, wcat, ne, bet):
    grid = ne // bet
    return pl.pallas_call(
        _edge_body,
        grid=(grid,),
        in_specs=[
            pl.BlockSpec((bet, 2 * C), lambda i: (i, 0)),
            pl.BlockSpec((D, bet), lambda i: (0, i)),
            pl.BlockSpec((D, 2 * C), lambda i: (0, 0)),
        ],
        out_specs=pl.BlockSpec((bet, C), lambda i: (i, 0)),
        out_shape=jax.ShapeDtypeStruct((ne, C), _F32),
    )(g, ewt, wcat)


_RPT = 624
_RTL = N - NS * _RPT
_ZR = 104


def _make_scatter_body(ne):
    epw = ne // NW
    nchunk = epw // BE
    npair = nchunk // 2
    assert nchunk % 2 == 0 and nchunk * BE == epw and epw % 8 == 0

    def body(m_hbm, dsti_hbm, out_hbm,
             idx0, idx1, mb0, mb1, stage, acc, sem0, sem1):
        c = lax.axis_index("c")
        s = lax.axis_index("s")
        wid = s * NC + c
        base = wid * epw
        rows0 = s * _RPT
        tail0 = NS * _RPT

        zero = jnp.zeros((16,), _F32)

        @pl.loop(0, _ZR)
        def _z(r):
            for k in range(C // 16):
                stage[r, pl.ds(k * 16, 16)] = zero

        for t in range(_RPT // _ZR):
            pltpu.sync_copy(stage, acc.at[pl.ds(rows0 + t * _ZR, _ZR)])

        @pl.when(s == NS - 1)
        def _():
            pltpu.sync_copy(stage.at[pl.ds(0, _RTL)], acc.at[pl.ds(tail0, _RTL)])

        plsc.subcore_barrier()

        ring = ((idx0, mb0, sem0), (idx1, mb1, sem1))
        for b in range(2):
            ib, mb, sm = ring[b]
            pltpu.async_copy(dsti_hbm.at[pl.ds(base + b * BE, BE)], ib, sm)
            pltpu.async_copy(m_hbm.at[pl.ds(base + b * BE, BE)], mb, sm)

        @pl.loop(0, npair)
        def _pair(t):
            for b in range(2):
                ib, mb, sm = ring[b]
                ch = 2 * t + b
                off = base + ch * BE
                pltpu.make_async_copy(dsti_hbm.at[pl.ds(off, BE)], ib, sm).wait()
                pltpu.make_async_copy(m_hbm.at[pl.ds(off, BE)], mb, sm).wait()
                pltpu.sync_copy(mb, acc.at[ib], add=True)

                @pl.when(ch + 2 < nchunk)
                def _():
                    off2 = base + (ch + 2) * BE
                    pltpu.async_copy(dsti_hbm.at[pl.ds(off2, BE)], ib, sm)
                    pltpu.async_copy(m_hbm.at[pl.ds(off2, BE)], mb, sm)

        plsc.subcore_barrier()

        for t in range(_RPT // _ZR):
            pltpu.sync_copy(acc.at[pl.ds(rows0 + t * _ZR, _ZR)], stage)
            pltpu.sync_copy(stage, out_hbm.at[pl.ds(c * N + rows0 + t * _ZR, _ZR)])

        @pl.when(s == NS - 1)
        def _():
            pltpu.sync_copy(acc.at[pl.ds(tail0, _RTL)], stage.at[pl.ds(0, _RTL)])
            pltpu.sync_copy(stage.at[pl.ds(0, _RTL)],
                            out_hbm.at[pl.ds(c * N + tail0, _RTL)])

    return body


def _make_sc_scatter(ne):
    body = _make_scatter_body(ne)
    mesh = plsc.VectorSubcoreMesh(core_axis_name="c", subcore_axis_name="s")
    return pl.kernel(
        body,
        out_type=jax.ShapeDtypeStruct((2 * N, C), _F32),
        mesh=mesh,
        scratch_types=[
            pltpu.VMEM((BE,), jnp.int32),
            pltpu.VMEM((BE,), jnp.int32),
            pltpu.VMEM((BE, C), _F32),
            pltpu.VMEM((BE, C), _F32),
            pltpu.VMEM((_ZR, C), _F32),
            pltpu.VMEM_SHARED((N, C), _F32),
            pltpu.SemaphoreType.DMA,
            pltpu.SemaphoreType.DMA,
        ],
    )


def _final_body(h_ref, a0_ref, a1_ref, a2_ref, a3_ref, a4_ref, a5_ref,
                wlt_ref, bl_ref, o_ref, acc_ref):
    i = pl.program_id(0)

    @pl.when(i == 0)
    def _():
        acc_ref[...] = jnp.zeros_like(acc_ref)

    h3 = (h_ref[...] + (a0_ref[...] + a1_ref[...]) + (a2_ref[...] + a3_ref[...])
          + (a4_ref[...] + a5_ref[...]))
    acc_ref[...] += jnp.sum(h3, axis=0, keepdims=True)

    @pl.when(i == pl.num_programs(0) - 1)
    def _():
        pooled = acc_ref[...] * (1.0 / N)
        o_ref[...] = jnp.dot(pooled, wlt_ref[...], preferred_element_type=_F32) + bl_ref[...]


def _final(h2, aggs, wlt, bl):
    grid = N // _BN
    return pl.pallas_call(
        _final_body,
        grid=(grid,),
        in_specs=[
            pl.BlockSpec((_BN, C), lambda i: (i, 0)),
            pl.BlockSpec((_BN, C), lambda i: (i, 0)),
            pl.BlockSpec((_BN, C), lambda i: (i + N // _BN, 0)),
            pl.BlockSpec((_BN, C), lambda i: (i, 0)),
            pl.BlockSpec((_BN, C), lambda i: (i + N // _BN, 0)),
            pl.BlockSpec((_BN, C), lambda i: (i, 0)),
            pl.BlockSpec((_BN, C), lambda i: (i + N // _BN, 0)),
            pl.BlockSpec((C, OUT), lambda i: (0, 0)),
            pl.BlockSpec((1, OUT), lambda i: (0, 0)),
        ],
        out_specs=pl.BlockSpec((1, OUT), lambda i: (0, 0)),
        out_shape=jax.ShapeDtypeStruct((1, OUT), _F32),
        scratch_shapes=[pltpu.VMEM((1, C), _F32)],
    )(h2, aggs[0], aggs[0], aggs[1], aggs[1], aggs[2], aggs[2], wlt, bl)


def _layer_weights(Wf, bf, Ws, bs):
    wd = jnp.concatenate([Wf[:, :C].T, Ws[:, :C].T], axis=1)
    wsr = jnp.concatenate([Wf[:, C:2 * C].T, Ws[:, C:2 * C].T], axis=1)
    bd = jnp.concatenate([bf, bs]).reshape(1, 2 * C)
    wcat = jnp.concatenate([Wf[:, 2 * C:].T, Ws[:, 2 * C:].T], axis=1)
    return wd, wsr, bd, wcat


def kernel(x, edge_index, edge_w, Wf1, bf1, Ws1, bs1, Wf2, bf2, Ws2, bs2,
           Wf3, bf3, Ws3, bs3, Wlin, blin):
    x = x.astype(_F32)
    edge_w = edge_w.astype(_F32)
    src = edge_index[0]
    dst = edge_index[1]
    ewt = edge_w.T

    offs = [0]
    for ne in SEGS:
        offs.append(offs[-1] + ne)
    dsts = [dst[offs[i]:offs[i + 1]] for i in range(len(SEGS))]
    srcs = [src[offs[i]:offs[i + 1]] for i in range(len(SEGS))]
    ewts = [ewt[:, offs[i]:offs[i + 1]] for i in range(len(SEGS))]

    gathers = {ne: _make_sc_gather(ne) for ne in set(SEGS)}
    scatters = {ne: _make_sc_scatter(ne) for ne in set(SEGS)}

    wds, wsrs, bds, wcats = zip(_layer_weights(Wf1, bf1, Ws1, bs1),
                                _layer_weights(Wf2, bf2, Ws2, bs2),
                                _layer_weights(Wf3, bf3, Ws3, bs3))

    h = x
    aggs = None
    for l in range(3):
        if l == 0:
            td, ts = _tables_first(x, wds[0], wsrs[0], bds[0])
        else:
            h, td, ts = _tables_res(h, aggs, wds[l], wsrs[l], bds[l])
        gs = []
        aggs = []
        gs.append(gathers[SEGS[0]](td, ts, dsts[0], srcs[0]))
        for i in range(len(SEGS)):
            if i + 1 < len(SEGS):
                gs.append(gathers[SEGS[i + 1]](td, ts, dsts[i + 1], srcs[i + 1]))
            m = _edge_tc(gs[i], ewts[i], wcats[l], SEGS[i], BET)
            aggs.append(scatters[SEGS[i]](m, dsts[i]))

    return _final(h, aggs, Wlin.T, blin.reshape(1, OUT))


# speedup vs baseline: 1.0433x; 1.0433x over previous
import functools

import jax
import jax.numpy as jnp
from jax import lax
from jax.experimental import pallas as pl
from jax.experimental.pallas import tpu as pltpu
from jax.experimental.pallas import tpu_sc as plsc

N = 10000
E = 320000
C = 128
D = 16
OUT = 64

NC = 2
NS = 16
NW = NC * NS
BE = 40

SEGS = (161280, 158720)
BET = 2560

_F32 = jnp.float32
_BF16 = jnp.bfloat16


def _tables_first_body(x_ref, wd_ref, ws_ref, bd_ref, td_ref, ts_ref):
    h = x_ref[...]
    td_ref[...] = jnp.dot(h, wd_ref[...], preferred_element_type=_F32) + bd_ref[...]
    ts_ref[...] = jnp.dot(h, ws_ref[...], preferred_element_type=_F32)


def _tables_res_body(x_ref, a0_ref, a1_ref, a2_ref, a3_ref,
                     wd_ref, ws_ref, bd_ref, h_ref, td_ref, ts_ref):
    h = jnp.maximum(x_ref[...] + (a0_ref[...] + a1_ref[...]) +
                    (a2_ref[...] + a3_ref[...]), 0.0)
    h_ref[...] = h
    td_ref[...] = jnp.dot(h, wd_ref[...], preferred_element_type=_F32) + bd_ref[...]
    ts_ref[...] = jnp.dot(h, ws_ref[...], preferred_element_type=_F32)


_BN = 1000


def _tables_first(x, wd, ws, bd):
    grid = N // _BN
    return pl.pallas_call(
        _tables_first_body,
        grid=(grid,),
        in_specs=[
            pl.BlockSpec((_BN, C), lambda i: (i, 0)),
            pl.BlockSpec((C, 2 * C), lambda i: (0, 0)),
            pl.BlockSpec((C, 2 * C), lambda i: (0, 0)),
            pl.BlockSpec((1, 2 * C), lambda i: (0, 0)),
        ],
        out_specs=[
            pl.BlockSpec((_BN, 2 * C), lambda i: (i, 0)),
            pl.BlockSpec((_BN, 2 * C), lambda i: (i, 0)),
        ],
        out_shape=[
            jax.ShapeDtypeStruct((N, 2 * C), _F32),
            jax.ShapeDtypeStruct((N, 2 * C), _F32),
        ],
    )(x, wd, ws, bd)


def _tables_res(x, aggs, wd, ws, bd):
    grid = N // _BN
    return pl.pallas_call(
        _tables_res_body,
        grid=(grid,),
        in_specs=[
            pl.BlockSpec((_BN, C), lambda i: (i, 0)),
            pl.BlockSpec((_BN, C), lambda i: (i, 0)),
            pl.BlockSpec((_BN, C), lambda i: (i + N // _BN, 0)),
            pl.BlockSpec((_BN, C), lambda i: (i, 0)),
            pl.BlockSpec((_BN, C), lambda i: (i + N // _BN, 0)),
            pl.BlockSpec((C, 2 * C), lambda i: (0, 0)),
            pl.BlockSpec((C, 2 * C), lambda i: (0, 0)),
            pl.BlockSpec((1, 2 * C), lambda i: (0, 0)),
        ],
        out_specs=[
            pl.BlockSpec((_BN, C), lambda i: (i, 0)),
            pl.BlockSpec((_BN, 2 * C), lambda i: (i, 0)),
            pl.BlockSpec((_BN, 2 * C), lambda i: (i, 0)),
        ],
        out_shape=[
            jax.ShapeDtypeStruct((N, C), _F32),
            jax.ShapeDtypeStruct((N, 2 * C), _F32),
            jax.ShapeDtypeStruct((N, 2 * C), _F32),
        ],
    )(x, aggs[0], aggs[0], aggs[1], aggs[1], wd, ws, bd)


def _make_gather_body(ne):
    epw = ne // NW
    nchunk = epw // BE
    npair = nchunk // 2
    assert nchunk % 2 == 0 and nchunk * BE == epw and epw % 8 == 0

    def body(td_hbm, ts_hbm, dsti_hbm, srci_hbm, out_hbm,
             idxad, idxas,
             bufd0, bufs0, obuf0, bufd1, bufs1, obuf1,
             semg0, semg1, semo0, semo1):
        c = lax.axis_index("c")
        s = lax.axis_index("s")
        wid = s * NC + c
        base = wid * epw

        pltpu.sync_copy(dsti_hbm.at[pl.ds(base, epw)], idxad)
        pltpu.sync_copy(srci_hbm.at[pl.ds(base, epw)], idxas)

        ring = ((bufd0, bufs0, obuf0, semg0, semo0),
                (bufd1, bufs1, obuf1, semg1, semo1))

        for b in range(2):
            bd, bs_, ob, sg, so = ring[b]
            pltpu.async_copy(td_hbm.at[idxad.at[pl.ds(b * BE, BE)]], bd, sg)
            pltpu.async_copy(ts_hbm.at[idxas.at[pl.ds(b * BE, BE)]], bs_, sg)

        @pl.loop(0, npair)
        def _pair(t):
            for b in range(2):
                bd, bs_, ob, sg, so = ring[b]
                ch = 2 * t + b
                off = base + ch * BE
                pltpu.make_async_copy(td_hbm.at[idxad.at[pl.ds(0, BE)]], bd, sg).wait()
                pltpu.make_async_copy(ts_hbm.at[idxas.at[pl.ds(0, BE)]], bs_, sg).wait()

                @pl.when(t > 0)
                def _():
                    pltpu.make_async_copy(ob, out_hbm.at[pl.ds(off, BE)], so).wait()

                @plsc.parallel_loop(0, BE, unroll=4)
                def _row(r):
                    for k in range(2 * C // 16):
                        ob[r, pl.ds(k * 16, 16)] = (bd[r, pl.ds(k * 16, 16)] +
                                                    bs_[r, pl.ds(k * 16, 16)])

                pltpu.async_copy(ob, out_hbm.at[pl.ds(off, BE)], so)

                @pl.when(ch + 2 < nchunk)
                def _():
                    off2 = (ch + 2) * BE
                    pltpu.async_copy(td_hbm.at[idxad.at[pl.ds(off2, BE)]], bd, sg)
                    pltpu.async_copy(ts_hbm.at[idxas.at[pl.ds(off2, BE)]], bs_, sg)

        for b in range(2):
            bd, bs_, ob, sg, so = ring[b]
            pltpu.make_async_copy(ob, out_hbm.at[pl.ds(base, BE)], so).wait()

    return body, epw


def _make_sc_gather(ne):
    body, epw = _make_gather_body(ne)
    mesh = plsc.VectorSubcoreMesh(core_axis_name="c", subcore_axis_name="s")
    kern = pl.kernel(
        body,
        out_type=jax.ShapeDtypeStruct((ne, 2 * C), _F32),
        mesh=mesh,
        scratch_types=[
            pltpu.VMEM((epw,), jnp.int32),
            pltpu.VMEM((epw,), jnp.int32),
            pltpu.VMEM((BE, 2 * C), _F32),
            pltpu.VMEM((BE, 2 * C), _F32),
            pltpu.VMEM((BE, 2 * C), _F32),
            pltpu.VMEM((BE, 2 * C), _F32),
            pltpu.VMEM((BE, 2 * C), _F32),
            pltpu.VMEM((BE, 2 * C), _F32),
            pltpu.SemaphoreType.DMA,
            pltpu.SemaphoreType.DMA,
            pltpu.SemaphoreType.DMA,
            pltpu.SemaphoreType.DMA,
        ],
    )
    return kern


def _edge_body(g_ref, ewt_ref, wcat_ref, m_ref):
    ep = lax.dot_general(ewt_ref[...], wcat_ref[...],
                         dimension_numbers=(((0,), (0,)), ((), ())),
                         preferred_element_type=_F32)
    g = g_ref[...].astype(_F32)
    fpre = g[:, :C] + ep[:, :C]
    spre = g[:, C:] + ep[:, C:]
    sig = 0.5 * jnp.tanh(0.5 * fpre) + 0.5
    sp = jnp.maximum(spre, 0.0) + jnp.log1p(jnp.exp(-jnp.abs(spre)))
    m_ref[...] = sig * sp


def _edge_tc(g, ewt, wcat, ne, bet):
    grid = ne // bet
    return pl.pallas_call(
        _edge_body,
        grid=(grid,),
        in_specs=[
            pl.BlockSpec((bet, 2 * C), lambda i: (i, 0)),
            pl.BlockSpec((D, bet), lambda i: (0, i)),
            pl.BlockSpec((D, 2 * C), lambda i: (0, 0)),
        ],
        out_specs=pl.BlockSpec((bet, C), lambda i: (i, 0)),
        out_shape=jax.ShapeDtypeStruct((ne, C), _F32),
    )(g, ewt, wcat)


_RPT = 624
_RTL = N - NS * _RPT
_ZR = 104


def _make_scatter_body(ne):
    epw = ne // NW
    nchunk = epw // BE
    npair = nchunk // 2
    assert nchunk % 2 == 0 and nchunk * BE == epw and epw % 8 == 0

    def body(m_hbm, dsti_hbm, out_hbm,
             idx0, idx1, mb0, mb1, stage, acc, sem0, sem1):
        c = lax.axis_index("c")
        s = lax.axis_index("s")
        wid = s * NC + c
        base = wid * epw
        rows0 = s * _RPT
        tail0 = NS * _RPT

        zero = jnp.zeros((16,), _F32)

        @pl.loop(0, _ZR)
        def _z(r):
            for k in range(C // 16):
                stage[r, pl.ds(k * 16, 16)] = zero

        for t in range(_RPT // _ZR):
            pltpu.sync_copy(stage, acc.at[pl.ds(rows0 + t * _ZR, _ZR)])

        @pl.when(s == NS - 1)
        def _():
            pltpu.sync_copy(stage.at[pl.ds(0, _RTL)], acc.at[pl.ds(tail0, _RTL)])

        plsc.subcore_barrier()

        ring = ((idx0, mb0, sem0), (idx1, mb1, sem1))
        for b in range(2):
            ib, mb, sm = ring[b]
            pltpu.async_copy(dsti_hbm.at[pl.ds(base + b * BE, BE)], ib, sm)
            pltpu.async_copy(m_hbm.at[pl.ds(base + b * BE, BE)], mb, sm)

        @pl.loop(0, npair)
        def _pair(t):
            for b in range(2):
                ib, mb, sm = ring[b]
                ch = 2 * t + b
                off = base + ch * BE
                pltpu.make_async_copy(dsti_hbm.at[pl.ds(off, BE)], ib, sm).wait()
                pltpu.make_async_copy(m_hbm.at[pl.ds(off, BE)], mb, sm).wait()
                pltpu.sync_copy(mb, acc.at[ib], add=True)

                @pl.when(ch + 2 < nchunk)
                def _():
                    off2 = base + (ch + 2) * BE
                    pltpu.async_copy(dsti_hbm.at[pl.ds(off2, BE)], ib, sm)
                    pltpu.async_copy(m_hbm.at[pl.ds(off2, BE)], mb, sm)

        plsc.subcore_barrier()

        for t in range(_RPT // _ZR):
            pltpu.sync_copy(acc.at[pl.ds(rows0 + t * _ZR, _ZR)], stage)
            pltpu.sync_copy(stage, out_hbm.at[pl.ds(c * N + rows0 + t * _ZR, _ZR)])

        @pl.when(s == NS - 1)
        def _():
            pltpu.sync_copy(acc.at[pl.ds(tail0, _RTL)], stage.at[pl.ds(0, _RTL)])
            pltpu.sync_copy(stage.at[pl.ds(0, _RTL)],
                            out_hbm.at[pl.ds(c * N + tail0, _RTL)])

    return body


def _make_sc_scatter(ne):
    body = _make_scatter_body(ne)
    mesh = plsc.VectorSubcoreMesh(core_axis_name="c", subcore_axis_name="s")
    return pl.kernel(
        body,
        out_type=jax.ShapeDtypeStruct((2 * N, C), _F32),
        mesh=mesh,
        scratch_types=[
            pltpu.VMEM((BE,), jnp.int32),
            pltpu.VMEM((BE,), jnp.int32),
            pltpu.VMEM((BE, C), _F32),
            pltpu.VMEM((BE, C), _F32),
            pltpu.VMEM((_ZR, C), _F32),
            pltpu.VMEM_SHARED((N, C), _F32),
            pltpu.SemaphoreType.DMA,
            pltpu.SemaphoreType.DMA,
        ],
    )


def _final_body(h_ref, a0_ref, a1_ref, a2_ref, a3_ref,
                wlt_ref, bl_ref, o_ref, acc_ref):
    i = pl.program_id(0)

    @pl.when(i == 0)
    def _():
        acc_ref[...] = jnp.zeros_like(acc_ref)

    h3 = h_ref[...] + (a0_ref[...] + a1_ref[...]) + (a2_ref[...] + a3_ref[...])
    acc_ref[...] += jnp.sum(h3, axis=0, keepdims=True)

    @pl.when(i == pl.num_programs(0) - 1)
    def _():
        pooled = acc_ref[...] * (1.0 / N)
        o_ref[...] = jnp.dot(pooled, wlt_ref[...], preferred_element_type=_F32) + bl_ref[...]


def _final(h2, aggs, wlt, bl):
    grid = N // _BN
    return pl.pallas_call(
        _final_body,
        grid=(grid,),
        in_specs=[
            pl.BlockSpec((_BN, C), lambda i: (i, 0)),
            pl.BlockSpec((_BN, C), lambda i: (i, 0)),
            pl.BlockSpec((_BN, C), lambda i: (i + N // _BN, 0)),
            pl.BlockSpec((_BN, C), lambda i: (i, 0)),
            pl.BlockSpec((_BN, C), lambda i: (i + N // _BN, 0)),
            pl.BlockSpec((C, OUT), lambda i: (0, 0)),
            pl.BlockSpec((1, OUT), lambda i: (0, 0)),
        ],
        out_specs=pl.BlockSpec((1, OUT), lambda i: (0, 0)),
        out_shape=jax.ShapeDtypeStruct((1, OUT), _F32),
        scratch_shapes=[pltpu.VMEM((1, C), _F32)],
    )(h2, aggs[0], aggs[0], aggs[1], aggs[1], wlt, bl)


def _layer_weights(Wf, bf, Ws, bs):
    wd = jnp.concatenate([Wf[:, :C].T, Ws[:, :C].T], axis=1)
    wsr = jnp.concatenate([Wf[:, C:2 * C].T, Ws[:, C:2 * C].T], axis=1)
    bd = jnp.concatenate([bf, bs]).reshape(1, 2 * C)
    wcat = jnp.concatenate([Wf[:, 2 * C:].T, Ws[:, 2 * C:].T], axis=1)
    return wd, wsr, bd, wcat


def kernel(x, edge_index, edge_w, Wf1, bf1, Ws1, bs1, Wf2, bf2, Ws2, bs2,
           Wf3, bf3, Ws3, bs3, Wlin, blin):
    x = x.astype(_F32)
    edge_w = edge_w.astype(_F32)
    src = edge_index[0]
    dst = edge_index[1]
    ewt = edge_w.T

    offs = [0]
    for ne in SEGS:
        offs.append(offs[-1] + ne)
    dsts = [dst[offs[i]:offs[i + 1]] for i in range(len(SEGS))]
    srcs = [src[offs[i]:offs[i + 1]] for i in range(len(SEGS))]
    ewts = [ewt[:, offs[i]:offs[i + 1]] for i in range(len(SEGS))]

    gathers = {ne: _make_sc_gather(ne) for ne in set(SEGS)}
    scatters = {ne: _make_sc_scatter(ne) for ne in set(SEGS)}

    wds, wsrs, bds, wcats = zip(_layer_weights(Wf1, bf1, Ws1, bs1),
                                _layer_weights(Wf2, bf2, Ws2, bs2),
                                _layer_weights(Wf3, bf3, Ws3, bs3))

    h = x
    aggs = None
    for l in range(3):
        if l == 0:
            td, ts = _tables_first(x, wds[0], wsrs[0], bds[0])
        else:
            h, td, ts = _tables_res(h, aggs, wds[l], wsrs[l], bds[l])
        gs = []
        aggs = []
        gs.append(gathers[SEGS[0]](td, ts, dsts[0], srcs[0]))
        for i in range(len(SEGS)):
            if i + 1 < len(SEGS):
                gs.append(gathers[SEGS[i + 1]](td, ts, dsts[i + 1], srcs[i + 1]))
            m = _edge_tc(gs[i], ewts[i], wcats[l], SEGS[i], BET)
            aggs.append(scatters[SEGS[i]](m, dsts[i]))

    return _final(h, aggs, Wlin.T, blin.reshape(1, OUT))


# speedup vs baseline: 1.0434x; 1.0001x over previous
import jax
import jax.numpy as jnp
from jax import lax
from jax.experimental import pallas as pl
from jax.experimental.pallas import tpu as pltpu
from jax.experimental.pallas import tpu_sc as plsc

N = 10000
E = 320000
C = 128
D = 16
OUT = 64

NC = 2
NS = 16
NW = NC * NS
BE = 40

SEGS = (161280, 158720)
BET = 2560

_F32 = jnp.float32


def _tables_first_body(x_ref, wd_ref, ws_ref, bd_ref, td_ref, ts_ref):
    h = x_ref[...]
    td_ref[...] = jnp.dot(h, wd_ref[...], preferred_element_type=_F32) + bd_ref[...]
    ts_ref[...] = jnp.dot(h, ws_ref[...], preferred_element_type=_F32)


def _tables_res_body(x_ref, a0_ref, a1_ref, a2_ref, a3_ref,
                     wd_ref, ws_ref, bd_ref, h_ref, td_ref, ts_ref):
    h = jnp.maximum(x_ref[...] + (a0_ref[...] + a1_ref[...]) +
                    (a2_ref[...] + a3_ref[...]), 0.0)
    h_ref[...] = h
    td_ref[...] = jnp.dot(h, wd_ref[...], preferred_element_type=_F32) + bd_ref[...]
    ts_ref[...] = jnp.dot(h, ws_ref[...], preferred_element_type=_F32)


_BN = 1000


def _tables_first(x, wd, ws, bd):
    grid = N // _BN
    return pl.pallas_call(
        _tables_first_body,
        grid=(grid,),
        in_specs=[
            pl.BlockSpec((_BN, C), lambda i: (i, 0)),
            pl.BlockSpec((C, 2 * C), lambda i: (0, 0)),
            pl.BlockSpec((C, 2 * C), lambda i: (0, 0)),
            pl.BlockSpec((1, 2 * C), lambda i: (0, 0)),
        ],
        out_specs=[
            pl.BlockSpec((_BN, 2 * C), lambda i: (i, 0)),
            pl.BlockSpec((_BN, 2 * C), lambda i: (i, 0)),
        ],
        out_shape=[
            jax.ShapeDtypeStruct((N, 2 * C), _F32),
            jax.ShapeDtypeStruct((N, 2 * C), _F32),
        ],
    )(x, wd, ws, bd)


def _tables_res(x, aggs, wd, ws, bd):
    grid = N // _BN
    return pl.pallas_call(
        _tables_res_body,
        grid=(grid,),
        in_specs=[
            pl.BlockSpec((_BN, C), lambda i: (i, 0)),
            pl.BlockSpec((_BN, C), lambda i: (i, 0)),
            pl.BlockSpec((_BN, C), lambda i: (i + N // _BN, 0)),
            pl.BlockSpec((_BN, C), lambda i: (i, 0)),
            pl.BlockSpec((_BN, C), lambda i: (i + N // _BN, 0)),
            pl.BlockSpec((C, 2 * C), lambda i: (0, 0)),
            pl.BlockSpec((C, 2 * C), lambda i: (0, 0)),
            pl.BlockSpec((1, 2 * C), lambda i: (0, 0)),
        ],
        out_specs=[
            pl.BlockSpec((_BN, C), lambda i: (i, 0)),
            pl.BlockSpec((_BN, 2 * C), lambda i: (i, 0)),
            pl.BlockSpec((_BN, 2 * C), lambda i: (i, 0)),
        ],
        out_shape=[
            jax.ShapeDtypeStruct((N, C), _F32),
            jax.ShapeDtypeStruct((N, 2 * C), _F32),
            jax.ShapeDtypeStruct((N, 2 * C), _F32),
        ],
    )(x, aggs[0], aggs[0], aggs[1], aggs[1], wd, ws, bd)


def _make_gather_body(ne):
    epw = ne // NW
    nchunk = epw // BE
    npair = nchunk // 2
    assert nchunk % 2 == 0 and nchunk * BE == epw and epw % 8 == 0

    def body(td_hbm, ts_hbm, dsti_hbm, srci_hbm, out_hbm,
             idxad, idxas,
             bufd0, bufs0, obuf0, bufd1, bufs1, obuf1,
             semg0, semg1, semo0, semo1):
        c = lax.axis_index("c")
        s = lax.axis_index("s")
        wid = s * NC + c
        base = wid * epw

        pltpu.sync_copy(dsti_hbm.at[pl.ds(base, epw)], idxad)
        pltpu.sync_copy(srci_hbm.at[pl.ds(base, epw)], idxas)

        ring = ((bufd0, bufs0, obuf0, semg0, semo0),
                (bufd1, bufs1, obuf1, semg1, semo1))

        for b in range(2):
            bd, bs_, ob, sg, so = ring[b]
            pltpu.async_copy(td_hbm.at[idxad.at[pl.ds(b * BE, BE)]], bd, sg)
            pltpu.async_copy(ts_hbm.at[idxas.at[pl.ds(b * BE, BE)]], bs_, sg)

        @pl.loop(0, npair)
        def _pair(t):
            for b in range(2):
                bd, bs_, ob, sg, so = ring[b]
                ch = 2 * t + b
                off = base + ch * BE
                pltpu.make_async_copy(td_hbm.at[idxad.at[pl.ds(0, BE)]], bd, sg).wait()
                pltpu.make_async_copy(ts_hbm.at[idxas.at[pl.ds(0, BE)]], bs_, sg).wait()

                @pl.when(t > 0)
                def _():
                    pltpu.make_async_copy(ob, out_hbm.at[pl.ds(off, BE)], so).wait()

                @plsc.parallel_loop(0, BE, unroll=4)
                def _row(r):
                    for k in range(2 * C // 16):
                        ob[r, pl.ds(k * 16, 16)] = (bd[r, pl.ds(k * 16, 16)] +
                                                    bs_[r, pl.ds(k * 16, 16)])

                pltpu.async_copy(ob, out_hbm.at[pl.ds(off, BE)], so)

                @pl.when(ch + 2 < nchunk)
                def _():
                    off2 = (ch + 2) * BE
                    pltpu.async_copy(td_hbm.at[idxad.at[pl.ds(off2, BE)]], bd, sg)
                    pltpu.async_copy(ts_hbm.at[idxas.at[pl.ds(off2, BE)]], bs_, sg)

        for b in range(2):
            bd, bs_, ob, sg, so = ring[b]
            pltpu.make_async_copy(ob, out_hbm.at[pl.ds(base, BE)], so).wait()

    return body, epw


def _make_sc_gather(ne):
    body, epw = _make_gather_body(ne)
    mesh = plsc.VectorSubcoreMesh(core_axis_name="c", subcore_axis_name="s")
    kern = pl.kernel(
        body,
        out_type=jax.ShapeDtypeStruct((ne, 2 * C), _F32),
        mesh=mesh,
        scratch_types=[
            pltpu.VMEM((epw,), jnp.int32),
            pltpu.VMEM((epw,), jnp.int32),
            pltpu.VMEM((BE, 2 * C), _F32),
            pltpu.VMEM((BE, 2 * C), _F32),
            pltpu.VMEM((BE, 2 * C), _F32),
            pltpu.VMEM((BE, 2 * C), _F32),
            pltpu.VMEM((BE, 2 * C), _F32),
            pltpu.VMEM((BE, 2 * C), _F32),
            pltpu.SemaphoreType.DMA,
            pltpu.SemaphoreType.DMA,
            pltpu.SemaphoreType.DMA,
            pltpu.SemaphoreType.DMA,
        ],
    )
    return kern


def _edge_body(g_ref, ewt_ref, wcat_ref, m_ref):
    ep = lax.dot_general(ewt_ref[...], wcat_ref[...],
                         dimension_numbers=(((0,), (0,)), ((), ())),
                         preferred_element_type=_F32)
    g = g_ref[...].astype(_F32)
    fpre = g[:, :C] + ep[:, :C]
    spre = g[:, C:] + ep[:, C:]
    sig = 0.5 * jnp.tanh(0.5 * fpre) + 0.5
    sp = jnp.maximum(spre, 0.0) + jnp.log1p(jnp.exp(-jnp.abs(spre)))
    m_ref[...] = sig * sp


def _edge_tc(g, ewt, wcat, ne, bet):
    grid = ne // bet
    return pl.pallas_call(
        _edge_body,
        grid=(grid,),
        in_specs=[
            pl.BlockSpec((bet, 2 * C), lambda i: (i, 0)),
            pl.BlockSpec((D, bet), lambda i: (0, i)),
            pl.BlockSpec((D, 2 * C), lambda i: (0, 0)),
        ],
        out_specs=pl.BlockSpec((bet, C), lambda i: (i, 0)),
        out_shape=jax.ShapeDtypeStruct((ne, C), _F32),
    )(g, ewt, wcat)


_RPT = 624
_RTL = N - NS * _RPT
_ZR = 104


def _make_scatter_body(ne):
    epw = ne // NW
    nchunk = epw // BE
    npair = nchunk // 2
    assert nchunk % 2 == 0 and nchunk * BE == epw and epw % 8 == 0

    def body(m_hbm, dsti_hbm, out_hbm,
             idx0, idx1, mb0, mb1, stage, acc, sem0, sem1):
        c = lax.axis_index("c")
        s = lax.axis_index("s")
        wid = s * NC + c
        base = wid * epw
        rows0 = s * _RPT
        tail0 = NS * _RPT

        zero = jnp.zeros((16,), _F32)

        @pl.loop(0, _ZR)
        def _z(r):
            for k in range(C // 16):
                stage[r, pl.ds(k * 16, 16)] = zero

        for t in range(_RPT // _ZR):
            pltpu.sync_copy(stage, acc.at[pl.ds(rows0 + t * _ZR, _ZR)])

        @pl.when(s == NS - 1)
        def _():
            pltpu.sync_copy(stage.at[pl.ds(0, _RTL)], acc.at[pl.ds(tail0, _RTL)])

        plsc.subcore_barrier()

        ring = ((idx0, mb0, sem0), (idx1, mb1, sem1))
        for b in range(2):
            ib, mb, sm = ring[b]
            pltpu.async_copy(dsti_hbm.at[pl.ds(base + b * BE, BE)], ib, sm)
            pltpu.async_copy(m_hbm.at[pl.ds(base + b * BE, BE)], mb, sm)

        @pl.loop(0, npair)
        def _pair(t):
            for b in range(2):
                ib, mb, sm = ring[b]
                ch = 2 * t + b
                off = base + ch * BE
                pltpu.make_async_copy(dsti_hbm.at[pl.ds(off, BE)], ib, sm).wait()
                pltpu.make_async_copy(m_hbm.at[pl.ds(off, BE)], mb, sm).wait()
                pltpu.sync_copy(mb, acc.at[ib], add=True)

                @pl.when(ch + 2 < nchunk)
                def _():
                    off2 = base + (ch + 2) * BE
                    pltpu.async_copy(dsti_hbm.at[pl.ds(off2, BE)], ib, sm)
                    pltpu.async_copy(m_hbm.at[pl.ds(off2, BE)], mb, sm)

        plsc.subcore_barrier()

        for t in range(_RPT // _ZR):
            pltpu.sync_copy(acc.at[pl.ds(rows0 + t * _ZR, _ZR)], stage)
            pltpu.sync_copy(stage, out_hbm.at[pl.ds(c * N + rows0 + t * _ZR, _ZR)])

        @pl.when(s == NS - 1)
        def _():
            pltpu.sync_copy(acc.at[pl.ds(tail0, _RTL)], stage.at[pl.ds(0, _RTL)])
            pltpu.sync_copy(stage.at[pl.ds(0, _RTL)],
                            out_hbm.at[pl.ds(c * N + tail0, _RTL)])

    return body


def _make_sc_scatter(ne):
    body = _make_scatter_body(ne)
    mesh = plsc.VectorSubcoreMesh(core_axis_name="c", subcore_axis_name="s")
    return pl.kernel(
        body,
        out_type=jax.ShapeDtypeStruct((2 * N, C), _F32),
        mesh=mesh,
        scratch_types=[
            pltpu.VMEM((BE,), jnp.int32),
            pltpu.VMEM((BE,), jnp.int32),
            pltpu.VMEM((BE, C), _F32),
            pltpu.VMEM((BE, C), _F32),
            pltpu.VMEM((_ZR, C), _F32),
            pltpu.VMEM_SHARED((N, C), _F32),
            pltpu.SemaphoreType.DMA,
            pltpu.SemaphoreType.DMA,
        ],
    )


def _final_body(h_ref, a0_ref, a1_ref, a2_ref, a3_ref,
                wlt_ref, bl_ref, o_ref, acc_ref):
    i = pl.program_id(0)

    @pl.when(i == 0)
    def _():
        acc_ref[...] = jnp.zeros_like(acc_ref)

    h3 = h_ref[...] + (a0_ref[...] + a1_ref[...]) + (a2_ref[...] + a3_ref[...])
    acc_ref[...] += jnp.sum(h3, axis=0, keepdims=True)

    @pl.when(i == pl.num_programs(0) - 1)
    def _():
        pooled = acc_ref[...] * (1.0 / N)
        o_ref[...] = jnp.dot(pooled, wlt_ref[...], preferred_element_type=_F32) + bl_ref[...]


def _final(h2, aggs, wlt, bl):
    grid = N // _BN
    return pl.pallas_call(
        _final_body,
        grid=(grid,),
        in_specs=[
            pl.BlockSpec((_BN, C), lambda i: (i, 0)),
            pl.BlockSpec((_BN, C), lambda i: (i, 0)),
            pl.BlockSpec((_BN, C), lambda i: (i + N // _BN, 0)),
            pl.BlockSpec((_BN, C), lambda i: (i, 0)),
            pl.BlockSpec((_BN, C), lambda i: (i + N // _BN, 0)),
            pl.BlockSpec((C, OUT), lambda i: (0, 0)),
            pl.BlockSpec((1, OUT), lambda i: (0, 0)),
        ],
        out_specs=pl.BlockSpec((1, OUT), lambda i: (0, 0)),
        out_shape=jax.ShapeDtypeStruct((1, OUT), _F32),
        scratch_shapes=[pltpu.VMEM((1, C), _F32)],
    )(h2, aggs[0], aggs[0], aggs[1], aggs[1], wlt, bl)


def _layer_weights(Wf, bf, Ws, bs):
    wd = jnp.concatenate([Wf[:, :C].T, Ws[:, :C].T], axis=1)
    wsr = jnp.concatenate([Wf[:, C:2 * C].T, Ws[:, C:2 * C].T], axis=1)
    bd = jnp.concatenate([bf, bs]).reshape(1, 2 * C)
    wcat = jnp.concatenate([Wf[:, 2 * C:].T, Ws[:, 2 * C:].T], axis=1)
    return wd, wsr, bd, wcat


def kernel(x, edge_index, edge_w, Wf1, bf1, Ws1, bs1, Wf2, bf2, Ws2, bs2,
           Wf3, bf3, Ws3, bs3, Wlin, blin):
    x = x.astype(_F32)
    edge_w = edge_w.astype(_F32)
    src = edge_index[0]
    dst = edge_index[1]
    ewt = edge_w.T

    offs = [0]
    for ne in SEGS:
        offs.append(offs[-1] + ne)
    dsts = [dst[offs[i]:offs[i + 1]] for i in range(len(SEGS))]
    srcs = [src[offs[i]:offs[i + 1]] for i in range(len(SEGS))]
    ewts = [ewt[:, offs[i]:offs[i + 1]] for i in range(len(SEGS))]

    gathers = {ne: _make_sc_gather(ne) for ne in set(SEGS)}
    scatters = {ne: _make_sc_scatter(ne) for ne in set(SEGS)}

    wds, wsrs, bds, wcats = zip(_layer_weights(Wf1, bf1, Ws1, bs1),
                                _layer_weights(Wf2, bf2, Ws2, bs2),
                                _layer_weights(Wf3, bf3, Ws3, bs3))

    h = x
    aggs = None
    for l in range(3):
        if l == 0:
            td, ts = _tables_first(x, wds[0], wsrs[0], bds[0])
        else:
            h, td, ts = _tables_res(h, aggs, wds[l], wsrs[l], bds[l])
        gs = []
        aggs = []
        gs.append(gathers[SEGS[0]](td, ts, dsts[0], srcs[0]))
        for i in range(len(SEGS)):
            if i + 1 < len(SEGS):
                gs.append(gathers[SEGS[i + 1]](td, ts, dsts[i + 1], srcs[i + 1]))
            m = _edge_tc(gs[i], ewts[i], wcats[l], SEGS[i], BET)
            aggs.append(scatters[SEGS[i]](m, dsts[i]))

    return _final(h, aggs, Wlin.T, blin.reshape(1, OUT))
